# baseline (speedup 1.0000x reference)
import functools

import jax
import jax.numpy as jnp
from jax import lax
from jax.experimental import pallas as pl
from jax.experimental.pallas import tpu as pltpu
from jax.experimental.pallas import tpu_sc as plsc

NUM_EMB = 100000
SEM_DIM = 4
EMB_DIM = 64
PAD = NUM_EMB * SEM_DIM

B, L, LF = 4096, 200, 4
NSEQ = B * L
NFUT = B * LF

NC, NS, LANES = 2, 16, 16
NW = NC * NS

SEQ_PER_W = NSEQ // NW
FUT_PER_W = NFUT // NW
CHUNK = 128
SEQ_CHUNKS = SEQ_PER_W // CHUNK
FUT_CHUNKS = FUT_PER_W // CHUNK
ALL_CHUNKS = SEQ_CHUNKS + FUT_CHUNKS
VEC_PER_CHUNK = CHUNK // LANES

K = 4
SET_ROWS = K * CHUNK
NGROUPS = ALL_CHUNKS // K
NPAIRS = SEQ_CHUNKS // (2 * K)

SLAB = 3200
NSLABS = SEQ_PER_W // SLAB
SLAB_CHUNKS = SLAB // CHUNK


def _idx_chunk(sem_v, tt_v, msk_v, idx_v, chunk_id, voff, use_mask):
  for u in range(VEC_PER_CHUNK):
    off = voff + u * LANES
    s = sem_v[pl.ds(off, LANES)]
    t = tt_v[pl.ds(off, LANES)]
    tc = jnp.clip(t, 0, SEM_DIM - 1)
    idx = tc * NUM_EMB + s
    valid = (s >= 0) & (s < NUM_EMB)
    idx = jnp.where(valid, idx, PAD)
    if use_mask:
      m = msk_v[pl.ds(off, LANES)]
      idx = jnp.where(m != 0, idx, PAD)
    idx_v[chunk_id, pl.ds(u * LANES, LANES)] = idx


def _sc_body(sem_h, tt_h, msk_h, semf_h, ttf_h, table_h,
             out_seq_h, out_fut_h,
             sem_v, tt_v, msk_v, idx_v, rows_a, rows_b, gsem, wsem):
  wid = lax.axis_index("s") * NC + lax.axis_index("c")
  base = wid * SEQ_PER_W
  basef = wid * FUT_PER_W

  def slab_loop(sidx, _):
    soff = base + sidx * SLAB
    pltpu.sync_copy(sem_h.at[pl.ds(soff, SLAB)], sem_v)
    pltpu.sync_copy(tt_h.at[pl.ds(soff, SLAB)], tt_v)
    pltpu.sync_copy(msk_h.at[pl.ds(soff, SLAB)], msk_v)

    def chunk_loop(c, _):
      _idx_chunk(sem_v, tt_v, msk_v, idx_v,
                 sidx * SLAB_CHUNKS + c, c * CHUNK, use_mask=True)
      return _
    lax.fori_loop(0, SLAB_CHUNKS, chunk_loop, 0)
    return _
  lax.fori_loop(0, NSLABS, slab_loop, 0)

  pltpu.sync_copy(semf_h.at[pl.ds(basef, FUT_PER_W)],
                  sem_v.at[pl.ds(0, FUT_PER_W)])
  pltpu.sync_copy(ttf_h.at[pl.ds(basef, FUT_PER_W)],
                  tt_v.at[pl.ds(0, FUT_PER_W)])
  for c in range(FUT_CHUNKS):
    _idx_chunk(sem_v, tt_v, msk_v, idx_v,
               SEQ_CHUNKS + c, c * CHUNK, use_mask=False)

  def fire_gathers(group, rows_set):
    for b in range(K):
      pltpu.async_copy(table_h.at[idx_v.at[group * K + b]],
                       rows_set.at[pl.ds(b * CHUNK, CHUNK)], gsem)

  def fire_seq_writes(group, rows_set):
    pltpu.async_copy(rows_set,
                     out_seq_h.at[pl.ds(base + group * SET_ROWS, SET_ROWS)],
                     wsem)

  def wait_gathers(rows_set):
    pltpu.make_async_copy(out_seq_h.at[pl.ds(0, SET_ROWS)], rows_set,
                          gsem).wait()

  def wait_writes(rows_set):
    pltpu.make_async_copy(rows_set, out_seq_h.at[pl.ds(0, SET_ROWS)],
                          wsem).wait()

  fire_gathers(0, rows_a)

  def pair_loop(g2, carry):
    g_a = 2 * g2

    @pl.when(g2 > 0)
    def _():
      wait_writes(rows_b)
    fire_gathers(g_a + 1, rows_b)
    wait_gathers(rows_a)
    fire_seq_writes(g_a, rows_a)
    wait_writes(rows_a)
    fire_gathers(g_a + 2, rows_a)
    wait_gathers(rows_b)
    fire_seq_writes(g_a + 1, rows_b)
    return carry
  lax.fori_loop(0, NPAIRS, pair_loop, 0)

  wait_writes(rows_b)
  wait_gathers(rows_a)
  pltpu.async_copy(rows_a, out_fut_h.at[pl.ds(basef, FUT_PER_W)], wsem)
  wait_writes(rows_a)


@jax.jit
def _run(sem_flat, tt_flat, msk_flat, semf_flat, ttf_flat, table):
  mesh = plsc.VectorSubcoreMesh(core_axis_name="c", subcore_axis_name="s",
                                num_cores=NC, num_subcores=NS)
  f = pl.kernel(
      _sc_body,
      out_type=[
          jax.ShapeDtypeStruct((NSEQ, EMB_DIM), jnp.float32),
          jax.ShapeDtypeStruct((NFUT, EMB_DIM), jnp.float32),
      ],
      mesh=mesh,
      scratch_types=[
          pltpu.VMEM((SLAB,), jnp.int32),
          pltpu.VMEM((SLAB,), jnp.int32),
          pltpu.VMEM((SLAB,), jnp.int32),
          pltpu.VMEM((ALL_CHUNKS, CHUNK), jnp.int32),
          pltpu.VMEM((SET_ROWS, EMB_DIM), jnp.float32),
          pltpu.VMEM((SET_ROWS, EMB_DIM), jnp.float32),
          pltpu.SemaphoreType.DMA,
          pltpu.SemaphoreType.DMA,
      ],
      compiler_params=pltpu.CompilerParams(use_tc_tiling_on_sc=False),
  )
  return f(sem_flat, tt_flat, msk_flat, semf_flat, ttf_flat, table)


def kernel(sem_ids, token_type_ids, seq_mask, sem_ids_fut, token_type_ids_fut,
           table):
  sem_flat = sem_ids.reshape(-1).astype(jnp.int32)
  tt_flat = token_type_ids.reshape(-1).astype(jnp.int32)
  msk_flat = seq_mask.reshape(-1).astype(jnp.int32)
  semf_flat = sem_ids_fut.reshape(-1).astype(jnp.int32)
  ttf_flat = token_type_ids_fut.reshape(-1).astype(jnp.int32)
  out_seq, out_fut = _run(sem_flat, tt_flat, msk_flat, semf_flat, ttf_flat,
                          table.astype(jnp.float32))
  return (out_seq.reshape(B, L, EMB_DIM), out_fut.reshape(B, LF, EMB_DIM))


# speedup vs baseline: 8.1330x; 8.1330x over previous
import functools

import jax
import jax.numpy as jnp
from jax import lax
from jax.experimental import pallas as pl
from jax.experimental.pallas import tpu as pltpu
from jax.experimental.pallas import tpu_sc as plsc

NUM_EMB = 100000
SEM_DIM = 4
EMB_DIM = 64
PAD = NUM_EMB * SEM_DIM

B, L, LF = 4096, 200, 4
NSEQ = B * L
NFUT = B * LF

NC, NS, LANES = 2, 16, 16
NW = NC * NS

SEQ_PER_W = NSEQ // NW
FUT_PER_W = NFUT // NW
CHUNK = 128
SEQ_CHUNKS = SEQ_PER_W // CHUNK
FUT_CHUNKS = FUT_PER_W // CHUNK
ALL_CHUNKS = SEQ_CHUNKS + FUT_CHUNKS
VEC_PER_CHUNK = CHUNK // LANES
COLV = EMB_DIM // LANES

K = 4
SET_ROWS = K * CHUNK
NPAIRS = SEQ_CHUNKS // (2 * K)

SLAB = 1280
NSLABS = SEQ_PER_W // SLAB
SLAB_CHUNKS = SLAB // CHUNK


def _idx_chunk(sem_v, tt_v, msk_v, idx_v, maskf_v, chunk_id, voff, use_mask):
  for u in range(VEC_PER_CHUNK):
    off = voff + u * LANES
    s = sem_v[pl.ds(off, LANES)]
    t = tt_v[pl.ds(off, LANES)]
    tc = jnp.clip(t, 0, SEM_DIM - 1)
    sc = jnp.clip(s, 0, NUM_EMB - 1)
    idx = tc * NUM_EMB + sc
    keep = (s >= 0) & (s < NUM_EMB)
    if use_mask:
      m = msk_v[pl.ds(off, LANES)]
      keep = keep & (m != 0)
    mf = jnp.where(keep, jnp.float32(1.0), jnp.float32(0.0))
    idx_v[chunk_id, pl.ds(u * LANES, LANES)] = idx
    maskf_v[chunk_id, pl.ds(u * LANES, LANES)] = mf


def _sc_body(sem_h, tt_h, msk_h, semf_h, ttf_h, table_h,
             out_seq_h, out_fut_h,
             sem_v, tt_v, msk_v, idx_v, maskf_v, rows_a, rows_b, gsem, wsem):
  wid = lax.axis_index("s") * NC + lax.axis_index("c")
  base = wid * SEQ_PER_W
  basef = wid * FUT_PER_W

  def slab_loop(sidx, carry):
    soff = base + sidx * SLAB
    pltpu.sync_copy(sem_h.at[pl.ds(soff, SLAB)], sem_v)
    pltpu.sync_copy(tt_h.at[pl.ds(soff, SLAB)], tt_v)
    pltpu.sync_copy(msk_h.at[pl.ds(soff, SLAB)], msk_v)

    def chunk_loop(c, inner):
      _idx_chunk(sem_v, tt_v, msk_v, idx_v, maskf_v,
                 sidx * SLAB_CHUNKS + c, c * CHUNK, use_mask=True)
      return inner
    lax.fori_loop(0, SLAB_CHUNKS, chunk_loop, 0)
    return carry
  lax.fori_loop(0, NSLABS, slab_loop, 0)

  pltpu.sync_copy(semf_h.at[pl.ds(basef, FUT_PER_W)],
                  sem_v.at[pl.ds(0, FUT_PER_W)])
  pltpu.sync_copy(ttf_h.at[pl.ds(basef, FUT_PER_W)],
                  tt_v.at[pl.ds(0, FUT_PER_W)])
  for c in range(FUT_CHUNKS):
    _idx_chunk(sem_v, tt_v, msk_v, idx_v, maskf_v,
               SEQ_CHUNKS + c, c * CHUNK, use_mask=False)

  def fire_gathers(group, rows_set):
    for b in range(K):
      pltpu.async_copy(table_h.at[idx_v.at[group * K + b]],
                       rows_set.at[pl.ds(b * CHUNK, CHUNK)], gsem)

  def mask_set(group, rows_set):
    for b in range(K):
      cid = group * K + b

      def row_loop(r16, carry):
        mv = maskf_v[cid, pl.ds(r16 * LANES, LANES)]
        for rr in range(LANES):
          m = mv[rr]
          roff = b * CHUNK + r16 * LANES + rr
          for c in range(COLV):
            v = rows_set[roff, pl.ds(c * LANES, LANES)]
            rows_set[roff, pl.ds(c * LANES, LANES)] = v * m
        return carry
      lax.fori_loop(0, CHUNK // LANES, row_loop, 0)

  def fire_seq_writes(group, rows_set):
    pltpu.async_copy(rows_set,
                     out_seq_h.at[pl.ds(base + group * SET_ROWS, SET_ROWS)],
                     wsem)

  def wait_gathers(rows_set):
    pltpu.make_async_copy(out_seq_h.at[pl.ds(0, SET_ROWS)], rows_set,
                          gsem).wait()

  def wait_writes(rows_set):
    pltpu.make_async_copy(rows_set, out_seq_h.at[pl.ds(0, SET_ROWS)],
                          wsem).wait()

  fire_gathers(0, rows_a)

  def pair_loop(g2, carry):
    g_a = 2 * g2

    @pl.when(g2 > 0)
    def _():
      wait_writes(rows_b)
    fire_gathers(g_a + 1, rows_b)
    wait_gathers(rows_a)
    mask_set(g_a, rows_a)
    fire_seq_writes(g_a, rows_a)
    wait_writes(rows_a)
    fire_gathers(g_a + 2, rows_a)
    wait_gathers(rows_b)
    mask_set(g_a + 1, rows_b)
    fire_seq_writes(g_a + 1, rows_b)
    return carry
  lax.fori_loop(0, NPAIRS, pair_loop, 0)

  wait_writes(rows_b)
  wait_gathers(rows_a)
  mask_set(SEQ_CHUNKS // K, rows_a)
  pltpu.async_copy(rows_a, out_fut_h.at[pl.ds(basef, FUT_PER_W)], wsem)
  wait_writes(rows_a)


@jax.jit
def _run(sem_flat, tt_flat, msk_flat, semf_flat, ttf_flat, table):
  mesh = plsc.VectorSubcoreMesh(core_axis_name="c", subcore_axis_name="s",
                                num_cores=NC, num_subcores=NS)
  f = pl.kernel(
      _sc_body,
      out_type=[
          jax.ShapeDtypeStruct((NSEQ, EMB_DIM), jnp.float32),
          jax.ShapeDtypeStruct((NFUT, EMB_DIM), jnp.float32),
      ],
      mesh=mesh,
      scratch_types=[
          pltpu.VMEM((SLAB,), jnp.int32),
          pltpu.VMEM((SLAB,), jnp.int32),
          pltpu.VMEM((SLAB,), jnp.int32),
          pltpu.VMEM((ALL_CHUNKS, CHUNK), jnp.int32),
          pltpu.VMEM((ALL_CHUNKS, CHUNK), jnp.float32),
          pltpu.VMEM((SET_ROWS, EMB_DIM), jnp.float32),
          pltpu.VMEM((SET_ROWS, EMB_DIM), jnp.float32),
          pltpu.SemaphoreType.DMA,
          pltpu.SemaphoreType.DMA,
      ],
      compiler_params=pltpu.CompilerParams(use_tc_tiling_on_sc=False),
  )
  return f(sem_flat, tt_flat, msk_flat, semf_flat, ttf_flat, table)


def kernel(sem_ids, token_type_ids, seq_mask, sem_ids_fut, token_type_ids_fut,
           table):
  sem_flat = sem_ids.reshape(-1).astype(jnp.int32)
  tt_flat = token_type_ids.reshape(-1).astype(jnp.int32)
  msk_flat = seq_mask.reshape(-1).astype(jnp.int32)
  semf_flat = sem_ids_fut.reshape(-1).astype(jnp.int32)
  ttf_flat = token_type_ids_fut.reshape(-1).astype(jnp.int32)
  out_seq, out_fut = _run(sem_flat, tt_flat, msk_flat, semf_flat, ttf_flat,
                          table.astype(jnp.float32))
  return (out_seq.reshape(B, L, EMB_DIM), out_fut.reshape(B, LF, EMB_DIM))
